# Initial kernel scaffold; baseline (speedup 1.0000x reference)
import jax
import jax.numpy as jnp
from jax.experimental import pallas as pl


def kernel(trainK, trainV, trainQ, trainVM, trainPM, trainKM, trainQM, inspect, A1):
    raise NotImplementedError("write your pallas kernel here")


# speedup vs baseline: 8.4299x; 8.4299x over previous
import functools

import jax
import jax.numpy as jnp
from jax import lax
from jax.experimental import pallas as pl
from jax.experimental.pallas import tpu as pltpu
from jax.experimental.pallas import tpu_sc as plsc

B, S, L, QL, E, V, OUT = 512, 50, 20, 30, 128, 100000, 1000
LANES = 16
NB = E // LANES
NC, NS = 2, 16
NW = NC * NS
B_PER_W = B // NW
SEG_PER_CH = 5
CH = S // SEG_PER_CH
CHROWS = SEG_PER_CH * L
QPAD = 32

W_K = [float(l) - (L - 1) / 2.0 for l in range(L)]
W_Q = [float(j) - (QL - 1) / 2.0 for j in range(QL)]


def _bag_body(a1, kidx, qidx, mem_out, q_out,
              kidx_v, qidx_v, krows, qrows, membuf, qbuf, sem0, sem1, semq):
    wid = lax.axis_index("s") * NC + lax.axis_index("c")
    b0 = wid * B_PER_W

    lane = lax.iota(jnp.int32, LANES).astype(jnp.float32)
    u_k = [(lane + (LANES * k - (E - 1) / 2.0)) * (4.0 / (E * L))
           for k in range(NB)]
    u_q = [(lane + (LANES * k - (E - 1) / 2.0)) * (4.0 / (E * QL))
           for k in range(NB)]

    def b_body(bi, carry):
        b = b0 + bi
        pltpu.sync_copy(kidx.at[b], kidx_v)
        pltpu.sync_copy(qidx.at[b], qidx_v)
        pltpu.async_copy(a1.at[kidx_v.at[0]], krows.at[0], sem0)
        pltpu.async_copy(a1.at[qidx_v], qrows, semq).wait()
        qacc0 = [jnp.zeros((LANES,), jnp.float32) for _ in range(NB)]
        qacc1 = [jnp.zeros((LANES,), jnp.float32) for _ in range(NB)]
        for j in range(QL):
            for k in range(NB):
                r = qrows[j, pl.ds(k * LANES, LANES)]
                qacc0[k] = qacc0[k] + r
                qacc1[k] = qacc1[k] + W_Q[j] * r
        for k in range(NB):
            qbuf[pl.ds(k * LANES, LANES)] = qacc0[k] + u_q[k] * qacc1[k]
        pltpu.sync_copy(qbuf, q_out.at[b])

        def compute_chunk(ph, c):
            def seg_body(si, _):
                acc0 = [jnp.zeros((LANES,), jnp.float32) for _ in range(NB)]
                acc1 = [jnp.zeros((LANES,), jnp.float32) for _ in range(NB)]
                base = si * L
                for l in range(L):
                    for k in range(NB):
                        r = krows[ph, base + l, pl.ds(k * LANES, LANES)]
                        acc0[k] = acc0[k] + r
                        acc1[k] = acc1[k] + W_K[l] * r
                srow = c * SEG_PER_CH + si
                for k in range(NB):
                    membuf[srow, pl.ds(k * LANES, LANES)] = (
                        acc0[k] + u_k[k] * acc1[k])
                return 0
            lax.fori_loop(0, SEG_PER_CH, seg_body, 0)

        def pair_body(p, _):
            c0 = 2 * p
            pltpu.make_async_copy(a1.at[kidx_v.at[c0]],
                                  krows.at[0], sem0).wait()
            pltpu.async_copy(a1.at[kidx_v.at[c0 + 1]], krows.at[1], sem1)
            compute_chunk(0, c0)
            pltpu.make_async_copy(a1.at[kidx_v.at[c0 + 1]],
                                  krows.at[1], sem1).wait()

            @pl.when(p < CH // 2 - 1)
            def _start_next():
                pltpu.async_copy(a1.at[kidx_v.at[c0 + 2]], krows.at[0], sem0)

            compute_chunk(1, c0 + 1)
            return 0

        lax.fori_loop(0, CH // 2, pair_body, 0)
        pltpu.sync_copy(membuf, mem_out.at[b])
        return 0

    lax.fori_loop(0, B_PER_W, b_body, 0)


_bag = functools.partial(
    pl.kernel,
    out_type=[jax.ShapeDtypeStruct((B, S, E), jnp.float32),
              jax.ShapeDtypeStruct((B, E), jnp.float32)],
    mesh=plsc.VectorSubcoreMesh(core_axis_name="c", subcore_axis_name="s"),
    scratch_types=[
        pltpu.VMEM((CH, CHROWS), jnp.int32),
        pltpu.VMEM((QPAD,), jnp.int32),
        pltpu.VMEM((2, CHROWS, E), jnp.float32),
        pltpu.VMEM((QPAD, E), jnp.float32),
        pltpu.VMEM((S, E), jnp.float32),
        pltpu.VMEM((E,), jnp.float32),
        pltpu.SemaphoreType.DMA,
        pltpu.SemaphoreType.DMA,
        pltpu.SemaphoreType.DMA,
    ],
)(_bag_body)


BB = 64


def _finish_body(mem_ref, q_ref, v_ref, pm_ref, y_ref, vi_ref, ap_ref):
    mem = mem_ref[...]
    q = q_ref[...]
    dot = jnp.sum(mem * q[:, None, :], axis=2)
    n1s = jnp.sum(mem * mem, axis=2)
    n2s = jnp.sum(q * q, axis=1, keepdims=True)
    scores = dot / jnp.maximum(jnp.sqrt(n1s * n2s), 1e-8)
    logits = scores + jnp.log(pm_ref[...] + 1e-45)
    m = jnp.max(logits, axis=1, keepdims=True)
    lse = jnp.log(jnp.sum(jnp.exp(logits - m), axis=1, keepdims=True))
    ap = logits - m - lse
    ap_ref[...] = ap
    po = jnp.max(ap, axis=1, keepdims=True)
    s_iota = lax.broadcasted_iota(jnp.int32, (BB, S), 1)
    idx = jnp.min(jnp.where(ap == po, s_iota, S), axis=1, keepdims=True)
    val = jnp.sum(jnp.where(s_iota == idx, v_ref[...], 0),
                  axis=1, keepdims=True)
    vi_ref[...] = val
    o_iota = lax.broadcasted_iota(jnp.int32, (BB, OUT), 1)
    y_ref[...] = jnp.where(o_iota == val, po, -100.0)


_finish = pl.pallas_call(
    _finish_body,
    grid=(B // BB,),
    in_specs=[
        pl.BlockSpec((BB, S, E), lambda i: (i, 0, 0)),
        pl.BlockSpec((BB, E), lambda i: (i, 0)),
        pl.BlockSpec((BB, S), lambda i: (i, 0)),
        pl.BlockSpec((BB, S), lambda i: (i, 0)),
    ],
    out_specs=[
        pl.BlockSpec((BB, OUT), lambda i: (i, 0)),
        pl.BlockSpec((BB, 1), lambda i: (i, 0)),
        pl.BlockSpec((BB, S), lambda i: (i, 0)),
    ],
    out_shape=[
        jax.ShapeDtypeStruct((B, OUT), jnp.float32),
        jax.ShapeDtypeStruct((B, 1), jnp.int32),
        jax.ShapeDtypeStruct((B, S), jnp.float32),
    ],
)


def kernel(trainK, trainV, trainQ, trainVM, trainPM, trainKM, trainQM,
           inspect, A1):
    kidx = trainK.reshape(B, CH, CHROWS).astype(jnp.int32)
    qidx = jnp.pad(trainQ.reshape(B, QL).astype(jnp.int32),
                   ((0, 0), (0, QPAD - QL)))
    mem, q = _bag(A1, kidx, qidx)
    y, vi, ap = _finish(mem, q, trainV, trainPM)
    return y, vi[:, 0], ap
